# baseline (speedup 1.0000x reference)
import jax
import jax.numpy as jnp
from jax import lax
from jax.experimental import pallas as pl
from jax.experimental.pallas import tpu as pltpu
from jax.experimental.pallas import tpu_sc as plsc

N_NODES = 10000
D_FEAT = 128
N_EDGES = 320000

NC = 2
NS = 16
NW = NC * NS
EPW = N_EDGES // NW
CH = 80
NCH = EPW // CH


def _sc_body(x_hbm, src_hbm, dst_hbm, out_hbm, sidx, didx, rows_v,
             acc_sh, gsem, isem, ssem):
    c = lax.axis_index("c")
    s = lax.axis_index("s")
    wid = s * NC + c

    def _zb(t, carry):
        rows_v[0, t // 8, pl.ds((t % 8) * 16, 16)] = jnp.zeros((16,),
                                                               jnp.float32)
        return carry

    lax.fori_loop(0, CH * 8, _zb, 0)

    @pl.when(s < 10)
    def _zero():
        for k in range(12):
            pltpu.async_copy(rows_v.at[0],
                             acc_sh.at[pl.ds(s * 1000 + k * CH, CH)], ssem)
        pltpu.async_copy(rows_v.at[0, pl.ds(0, 40)],
                         acc_sh.at[pl.ds(s * 1000 + 960, 40)], ssem)

    def _idx_fetch(i):
        b = i % 4
        pltpu.async_copy(src_hbm.at[wid, i], sidx.at[b], isem)
        pltpu.async_copy(dst_hbm.at[wid, i], didx.at[b], isem)

    def _idx_wait(i):
        b = i % 4
        pltpu.make_async_copy(src_hbm.at[wid, i], sidx.at[b], isem).wait()
        pltpu.make_async_copy(dst_hbm.at[wid, i], didx.at[b], isem).wait()

    def _gather(i):
        pltpu.async_copy(x_hbm.at[sidx.at[i % 4]], rows_v.at[i % 3], gsem)

    def _gather_wait(i):
        pltpu.make_async_copy(x_hbm.at[sidx.at[i % 4]], rows_v.at[i % 3],
                              gsem).wait()

    def _scatter(i):
        pltpu.async_copy(rows_v.at[i % 3], acc_sh.at[didx.at[i % 4]], ssem,
                         add=True)

    def _scatter_wait(i):
        pltpu.make_async_copy(rows_v.at[i % 3], acc_sh.at[didx.at[i % 4]],
                              ssem).wait()

    for i in range(3):
        _idx_fetch(i)

    @pl.when(s < 10)
    def _zero_wait():
        for k in range(12):
            pltpu.make_async_copy(
                rows_v.at[0], acc_sh.at[pl.ds(s * 1000 + k * CH, CH)],
                ssem).wait()
        pltpu.make_async_copy(
            rows_v.at[0, pl.ds(0, 40)],
            acc_sh.at[pl.ds(s * 1000 + 960, 40)], ssem).wait()

    _idx_wait(0)
    _gather(0)
    _idx_wait(1)
    _gather(1)
    plsc.subcore_barrier()

    def _step(i, carry):
        _gather_wait(i)

        @pl.when(i >= 1)
        def _drain():
            _scatter_wait(i - 1)

        @pl.when(i + 2 < NCH)
        def _next():
            _idx_wait(i + 2)
            _gather(i + 2)

        _scatter(i)

        @pl.when(i + 3 < NCH)
        def _prefetch():
            _idx_fetch(i + 3)

        return carry

    lax.fori_loop(0, NCH, _step, 0)
    _scatter_wait(NCH - 1)
    plsc.subcore_barrier()

    @pl.when(s < 10)
    def _out():
        pltpu.sync_copy(acc_sh.at[pl.ds(s * 1000, 1000)],
                        out_hbm.at[c, pl.ds(s * 1000, 1000)])


_sc_fused = pl.kernel(
    _sc_body,
    out_type=jax.ShapeDtypeStruct((NC, N_NODES, D_FEAT), jnp.float32),
    mesh=plsc.VectorSubcoreMesh(core_axis_name="c", subcore_axis_name="s",
                                num_cores=NC, num_subcores=NS),
    compiler_params=pltpu.CompilerParams(skip_device_barrier=True),
    scratch_types=[
        pltpu.VMEM((4, CH), jnp.int32),
        pltpu.VMEM((4, CH), jnp.int32),
        pltpu.VMEM((3, CH, D_FEAT), jnp.float32),
        pltpu.VMEM_SHARED((N_NODES, D_FEAT), jnp.float32),
        pltpu.SemaphoreType.DMA,
        pltpu.SemaphoreType.DMA,
        pltpu.SemaphoreType.DMA,
    ],
)


def _sum_body(p_ref, o_ref):
    o_ref[...] = p_ref[0] + p_ref[1]


def _tc_sum(partials):
    blk = 1000
    return pl.pallas_call(
        _sum_body,
        out_shape=jax.ShapeDtypeStruct((N_NODES, D_FEAT), jnp.float32),
        grid=(N_NODES // blk,),
        in_specs=[pl.BlockSpec((NC, blk, D_FEAT), lambda i: (0, i, 0))],
        out_specs=pl.BlockSpec((blk, D_FEAT), lambda i: (i, 0)),
    )(partials)


@jax.jit
def kernel(x, edge_index):
    src = edge_index[0].reshape(NW, NCH, CH)
    dst = edge_index[1].reshape(NW, NCH, CH)
    partials = _sc_fused(x, src, dst)
    return _tc_sum(partials)
